# baseline (speedup 1.0000x reference)
import functools

import jax
import jax.numpy as jnp
from jax import lax
from jax.experimental import pallas as pl
from jax.experimental.pallas import tpu as pltpu
from jax.experimental.pallas import tpu_sc as plsc

_DECAY = 0.95
_THRESH = 0.0001
_C = 4
_G = 2097152
_N = 524288
_CH = 16384
_NPAIR = _N // (2 * _CH)
_FB = 16384
_PER_CORE = 2 * _G
_FPT = _PER_CORE // 16
_FROUNDS = _FPT // _FB

_mesh = plsc.VectorSubcoreMesh(core_axis_name="c", subcore_axis_name="s")


@functools.partial(
    pl.kernel,
    mesh=_mesh,
    out_type=jax.ShapeDtypeStruct((_C * _G,), jnp.float32),
    scratch_types=[
        pltpu.VMEM((_FB,), jnp.float32),
        pltpu.VMEM((_CH,), jnp.int32),
        pltpu.VMEM((_CH,), jnp.int32),
        pltpu.VMEM((_CH,), jnp.float32),
        pltpu.VMEM((_CH,), jnp.float32),
        pltpu.SemaphoreType.DMA,
        pltpu.SemaphoreType.DMA,
        pltpu.SemaphoreType.DMA,
        pltpu.SemaphoreType.DMA,
        pltpu.SemaphoreType.DMA,
    ],
)
def _sc_scatter(idx_hbm, dens_hbm, tmp_hbm,
                fbuf, idx_a, idx_b, den_a, den_b,
                sia, sib, sda, sdb, ss):
    core = lax.axis_index("c")
    sub = lax.axis_index("s")

    def fill_vreg(i, c2):
        fbuf[pl.ds(i * 16, 16)] = jnp.full((16,), -1.0, jnp.float32)
        return c2

    lax.fori_loop(0, _FB // 16, fill_vreg, 0, unroll=8)
    base = core * _PER_CORE + sub * _FPT

    def fill_round(r, c2):
        pltpu.async_copy(fbuf, tmp_hbm.at[pl.ds(base + r * _FB, _FB)], ss)
        return c2

    lax.fori_loop(0, _FROUNDS, fill_round, 0)

    def fill_drain(r, c2):
        pltpu.make_async_copy(
            fbuf, tmp_hbm.at[pl.ds(base + r * _FB, _FB)], ss).wait()
        return c2

    lax.fori_loop(0, _FROUNDS, fill_drain, 0)

    plsc.subcore_barrier()

    @pl.when(sub < 2)
    def _():
        casc = core * 2 + sub
        goff = casc * _G
        soff = casc * _N

        def ld(buf_i, buf_d, chunk, si, sd):
            s0 = soff + chunk * _CH
            pltpu.async_copy(idx_hbm.at[pl.ds(s0, _CH)], buf_i, si)
            pltpu.async_copy(dens_hbm.at[pl.ds(s0, _CH)], buf_d, sd)

        def ld_wait(buf_i, buf_d, chunk, si, sd):
            s0 = soff + chunk * _CH
            pltpu.make_async_copy(idx_hbm.at[pl.ds(s0, _CH)], buf_i, si).wait()
            pltpu.make_async_copy(dens_hbm.at[pl.ds(s0, _CH)], buf_d, sd).wait()

        def offs(buf_i):
            def go(i, c2):
                sl = pl.ds(i * 16, 16)
                buf_i[sl] = buf_i[sl] + goff
                return c2
            lax.fori_loop(0, _CH // 16, go, 0, unroll=8)

        ld(idx_a, den_a, 0, sia, sda)

        def pair(kk, c2):
            e = 2 * kk
            o = e + 1
            ld_wait(idx_a, den_a, e, sia, sda)
            offs(idx_a)

            @pl.when(kk > 0)
            def _():
                pltpu.make_async_copy(den_b, tmp_hbm.at[idx_b], ss).wait()

            ld(idx_b, den_b, o, sib, sdb)
            pltpu.async_copy(den_a, tmp_hbm.at[idx_a], ss)
            ld_wait(idx_b, den_b, o, sib, sdb)
            offs(idx_b)
            pltpu.make_async_copy(den_a, tmp_hbm.at[idx_a], ss).wait()

            @pl.when(kk < _NPAIR - 1)
            def _():
                ld(idx_a, den_a, e + 2, sia, sda)

            pltpu.async_copy(den_b, tmp_hbm.at[idx_b], ss)
            return c2

        lax.fori_loop(0, _NPAIR, pair, 0)
        pltpu.make_async_copy(den_b, tmp_hbm.at[idx_b], ss).wait()


def _merge_body(grid_ref, tmp_ref, out_ref, sum_ref, cnt_ref):
    @pl.when(pl.program_id(0) == 0)
    def _():
        sum_ref[0, 0] = jnp.float32(0.0)
        cnt_ref[0, 0] = jnp.float32(0.0)

    g = grid_ref[...]
    t = tmp_ref[...]
    ng = jnp.where((g >= 0) & (t >= 0), jnp.maximum(g * _DECAY, t), g)
    out_ref[...] = ng
    pos = ng > 0
    sum_ref[0, 0] += jnp.sum(jnp.where(pos, ng, 0.0))
    cnt_ref[0, 0] += jnp.sum(pos.astype(jnp.float32))


def _bitfield_body(thr_ref, x_ref, out_ref):
    thr = thr_ref[0, 0]
    bits = (x_ref[...] > thr).astype(jnp.float32)
    row = lax.broadcasted_iota(jnp.int32, (1024, 128), 0)
    col = lax.broadcasted_iota(jnp.int32, (1024, 128), 1)
    w = jnp.where(row // 8 == col,
                  (1 << (row % 8)), 0).astype(jnp.float32)
    byte = jax.lax.dot(bits, w, preferred_element_type=jnp.float32)
    out_ref[...] = byte.astype(jnp.int32).astype(jnp.uint8)


def kernel(density_grid, indices, densities):
    idx_flat = indices.reshape(-1)
    dens_flat = densities.reshape(-1)

    tmp_flat = _sc_scatter(idx_flat, dens_flat)

    g2 = density_grid.reshape(8192, 1024)
    t2 = tmp_flat.reshape(8192, 1024)
    new2, s, c = pl.pallas_call(
        _merge_body,
        out_shape=(jax.ShapeDtypeStruct((8192, 1024), jnp.float32),
                   jax.ShapeDtypeStruct((1, 1), jnp.float32),
                   jax.ShapeDtypeStruct((1, 1), jnp.float32)),
        grid=(16,),
        in_specs=[pl.BlockSpec((512, 1024), lambda i: (i, 0)),
                  pl.BlockSpec((512, 1024), lambda i: (i, 0))],
        out_specs=(pl.BlockSpec((512, 1024), lambda i: (i, 0)),
                   pl.BlockSpec(memory_space=pltpu.SMEM),
                   pl.BlockSpec(memory_space=pltpu.SMEM)),
    )(g2, t2)
    mean_density = (s[0, 0] / jnp.maximum(c[0, 0], 1.0)).astype(jnp.float32)
    thr = jnp.minimum(mean_density, jnp.float32(_THRESH)).reshape(1, 1)

    bitfield = pl.pallas_call(
        _bitfield_body,
        out_shape=jax.ShapeDtypeStruct((8192, 128), jnp.uint8),
        grid=(16,),
        in_specs=[
            pl.BlockSpec(memory_space=pltpu.SMEM),
            pl.BlockSpec((512, 1024), lambda i: (i, 0)),
        ],
        out_specs=pl.BlockSpec((512, 128), lambda i: (i, 0)),
    )(thr, new2)

    return (new2.reshape(_C, _G),
            bitfield.reshape(-1),
            mean_density)


# speedup vs baseline: 1.0006x; 1.0006x over previous
import functools

import jax
import jax.numpy as jnp
from jax import lax
from jax.experimental import pallas as pl
from jax.experimental.pallas import tpu as pltpu
from jax.experimental.pallas import tpu_sc as plsc

_DECAY = 0.95
_THRESH = 0.0001
_C = 4
_G = 2097152
_N = 524288
_CH = 16384
_NPAIR = _N // (2 * _CH)
_FB = 16384
_PER_CORE = 2 * _G
_FPT = _PER_CORE // 16
_FROUNDS = _FPT // _FB

@functools.cache
def _sc_scatter_fn():
    mesh = plsc.VectorSubcoreMesh(core_axis_name="c", subcore_axis_name="s")
    return pl.kernel(
        _sc_scatter_body,
        mesh=mesh,
        out_type=jax.ShapeDtypeStruct((_C * _G,), jnp.float32),
        scratch_types=[
            pltpu.VMEM((_FB,), jnp.float32),
            pltpu.VMEM((_CH,), jnp.int32),
            pltpu.VMEM((_CH,), jnp.int32),
            pltpu.VMEM((_CH,), jnp.float32),
            pltpu.VMEM((_CH,), jnp.float32),
            pltpu.SemaphoreType.DMA,
            pltpu.SemaphoreType.DMA,
            pltpu.SemaphoreType.DMA,
            pltpu.SemaphoreType.DMA,
            pltpu.SemaphoreType.DMA,
        ],
    )


def _sc_scatter_body(idx_hbm, dens_hbm, tmp_hbm,
                     fbuf, idx_a, idx_b, den_a, den_b,
                     sia, sib, sda, sdb, ss):
    core = lax.axis_index("c")
    sub = lax.axis_index("s")

    def fill_vreg(i, c2):
        fbuf[pl.ds(i * 16, 16)] = jnp.full((16,), -1.0, jnp.float32)
        return c2

    lax.fori_loop(0, _FB // 16, fill_vreg, 0, unroll=8)
    base = core * _PER_CORE + sub * _FPT

    def fill_round(r, c2):
        pltpu.async_copy(fbuf, tmp_hbm.at[pl.ds(base + r * _FB, _FB)], ss)
        return c2

    lax.fori_loop(0, _FROUNDS, fill_round, 0)

    def fill_drain(r, c2):
        pltpu.make_async_copy(
            fbuf, tmp_hbm.at[pl.ds(base + r * _FB, _FB)], ss).wait()
        return c2

    lax.fori_loop(0, _FROUNDS, fill_drain, 0)

    plsc.subcore_barrier()

    @pl.when(sub < 2)
    def _():
        casc = core * 2 + sub
        goff = casc * _G
        soff = casc * _N

        def ld(buf_i, buf_d, chunk, si, sd):
            s0 = soff + chunk * _CH
            pltpu.async_copy(idx_hbm.at[pl.ds(s0, _CH)], buf_i, si)
            pltpu.async_copy(dens_hbm.at[pl.ds(s0, _CH)], buf_d, sd)

        def ld_wait(buf_i, buf_d, chunk, si, sd):
            s0 = soff + chunk * _CH
            pltpu.make_async_copy(idx_hbm.at[pl.ds(s0, _CH)], buf_i, si).wait()
            pltpu.make_async_copy(dens_hbm.at[pl.ds(s0, _CH)], buf_d, sd).wait()

        def offs(buf_i):
            def go(i, c2):
                sl = pl.ds(i * 16, 16)
                buf_i[sl] = buf_i[sl] + goff
                return c2
            lax.fori_loop(0, _CH // 16, go, 0, unroll=8)

        ld(idx_a, den_a, 0, sia, sda)

        def pair(kk, c2):
            e = 2 * kk
            o = e + 1
            ld_wait(idx_a, den_a, e, sia, sda)
            offs(idx_a)

            @pl.when(kk > 0)
            def _():
                pltpu.make_async_copy(den_b, tmp_hbm.at[idx_b], ss).wait()

            ld(idx_b, den_b, o, sib, sdb)
            pltpu.async_copy(den_a, tmp_hbm.at[idx_a], ss)
            ld_wait(idx_b, den_b, o, sib, sdb)
            offs(idx_b)
            pltpu.make_async_copy(den_a, tmp_hbm.at[idx_a], ss).wait()

            @pl.when(kk < _NPAIR - 1)
            def _():
                ld(idx_a, den_a, e + 2, sia, sda)

            pltpu.async_copy(den_b, tmp_hbm.at[idx_b], ss)
            return c2

        lax.fori_loop(0, _NPAIR, pair, 0)
        pltpu.make_async_copy(den_b, tmp_hbm.at[idx_b], ss).wait()


def _merge_body(grid_ref, tmp_ref, out_ref, sum_ref, cnt_ref):
    @pl.when(pl.program_id(0) == 0)
    def _():
        sum_ref[0, 0] = jnp.float32(0.0)
        cnt_ref[0, 0] = jnp.float32(0.0)

    g = grid_ref[...]
    t = tmp_ref[...]
    ng = jnp.where((g >= 0) & (t >= 0), jnp.maximum(g * _DECAY, t), g)
    out_ref[...] = ng
    pos = ng > 0
    sum_ref[0, 0] += jnp.sum(jnp.where(pos, ng, 0.0))
    cnt_ref[0, 0] += jnp.sum(pos.astype(jnp.float32))


def _bitfield_body(thr_ref, x_ref, out_ref):
    thr = thr_ref[0, 0]
    bits = (x_ref[...] > thr).astype(jnp.float32)
    row = lax.broadcasted_iota(jnp.int32, (1024, 128), 0)
    col = lax.broadcasted_iota(jnp.int32, (1024, 128), 1)
    w = jnp.where(row // 8 == col,
                  (1 << (row % 8)), 0).astype(jnp.float32)
    byte = jax.lax.dot(bits, w, preferred_element_type=jnp.float32)
    out_ref[...] = byte.astype(jnp.int32).astype(jnp.uint8)


def kernel(density_grid, indices, densities):
    idx_flat = indices.reshape(-1)
    dens_flat = densities.reshape(-1)

    tmp_flat = _sc_scatter_fn()(idx_flat, dens_flat)

    g2 = density_grid.reshape(8192, 1024)
    t2 = tmp_flat.reshape(8192, 1024)
    new2, s, c = pl.pallas_call(
        _merge_body,
        out_shape=(jax.ShapeDtypeStruct((8192, 1024), jnp.float32),
                   jax.ShapeDtypeStruct((1, 1), jnp.float32),
                   jax.ShapeDtypeStruct((1, 1), jnp.float32)),
        grid=(16,),
        in_specs=[pl.BlockSpec((512, 1024), lambda i: (i, 0)),
                  pl.BlockSpec((512, 1024), lambda i: (i, 0))],
        out_specs=(pl.BlockSpec((512, 1024), lambda i: (i, 0)),
                   pl.BlockSpec(memory_space=pltpu.SMEM),
                   pl.BlockSpec(memory_space=pltpu.SMEM)),
    )(g2, t2)
    mean_density = (s[0, 0] / jnp.maximum(c[0, 0], 1.0)).astype(jnp.float32)
    thr = jnp.minimum(mean_density, jnp.float32(_THRESH)).reshape(1, 1)

    bitfield = pl.pallas_call(
        _bitfield_body,
        out_shape=jax.ShapeDtypeStruct((8192, 128), jnp.uint8),
        grid=(16,),
        in_specs=[
            pl.BlockSpec(memory_space=pltpu.SMEM),
            pl.BlockSpec((512, 1024), lambda i: (i, 0)),
        ],
        out_specs=pl.BlockSpec((512, 128), lambda i: (i, 0)),
    )(thr, new2)

    return (new2.reshape(_C, _G),
            bitfield.reshape(-1),
            mean_density)
